# Initial kernel scaffold; baseline (speedup 1.0000x reference)
import jax
import jax.numpy as jnp
from jax.experimental import pallas as pl


def kernel(x_code, type_ids, edges_AST, edges_CFG, edges_DFG, edges_NCS, type_emb, in_proj_W, in_proj_b, rel_AST_W, rel_AST_b, rel_CFG_W, rel_CFG_b, rel_DFG_W, rel_DFG_b, rel_NCS_W, rel_NCS_b, gru_W_ih, gru_b_ih, gru_W_hh, gru_b_hh, conv1_W, conv1_b, conv2_W, conv2_b, fc1_W, fc1_b, fc2_W, fc2_b):
    raise NotImplementedError("write your pallas kernel here")


# speedup vs baseline: 2.1953x; 2.1953x over previous
import functools

import jax
import jax.numpy as jnp
from jax import lax
from jax.experimental import pallas as pl
from jax.experimental.pallas import tpu as pltpu
from jax.experimental.pallas import tpu_sc as plsc

N = 10000
NP = 10240
E = 80000
W2V = 128
TYPE_DIM = 32
HID = 128
NUM_TYPES = 100
TIME_STEPS = 6
NRELS = 4
BLK = 128

NC, NS = 2, 16
NTILES = NC * NS
EPT = E // NTILES
CH = 128
NCHUNK = (EPT + CH - 1) // CH
EPT_PAD = NCHUNK * CH
TRASH = N
RPT = NP // NS

_f32 = jnp.float32


def _sc_body(y0, y1, y2, y3, sidx, didx, m0, m1, acc, sbuf, dbuf, gbuf, zb, sem):
    c = lax.axis_index("c")
    s = lax.axis_index("s")
    g = c * NS + s
    base = s * RPT

    def _zfill(r, _):
        for l in range(HID // 16):
            zb[r, pl.ds(l * 16, 16)] = jnp.zeros((16,), _f32)
        return 0
    lax.fori_loop(0, 64, _zfill, 0)

    def _zcopy(k, _):
        pltpu.sync_copy(zb, acc.at[pl.ds(base + k * 64, 64)])
        return 0
    lax.fori_loop(0, RPT // 64, _zcopy, 0)

    for r in range(NRELS):
        pltpu.sync_copy(sidx.at[r, g], sbuf.at[pl.ds(r * NCHUNK, NCHUNK)])
        pltpu.sync_copy(didx.at[r, g], dbuf.at[pl.ds(r * NCHUNK, NCHUNK)])

    plsc.subcore_barrier()

    ys = [y0, y1, y2, y3]
    for r in range(NRELS):
        yr = ys[r]

        def _chunk(j, _, yr=yr, r=r):
            row = r * NCHUNK + j
            pltpu.async_copy(yr.at[sbuf.at[row]], gbuf, sem).wait()
            pltpu.sync_copy(gbuf, acc.at[dbuf.at[row]], add=True)
            return 0
        lax.fori_loop(0, NCHUNK, _chunk, 0)

    plsc.subcore_barrier()

    rows = pl.ds(base, RPT)

    @pl.when(c == 0)
    def _():
        pltpu.sync_copy(acc.at[rows], m0.at[rows])

    @pl.when(c == 1)
    def _():
        pltpu.sync_copy(acc.at[rows], m1.at[rows])


@functools.cache
def _get_sc_msg():
    mesh = plsc.VectorSubcoreMesh(
        core_axis_name="c", subcore_axis_name="s",
        num_cores=NC, num_subcores=NS)
    return pl.kernel(
        _sc_body,
        out_type=[jax.ShapeDtypeStruct((NP, HID), _f32),
                  jax.ShapeDtypeStruct((NP, HID), _f32)],
        mesh=mesh,
        scratch_types=[
            pltpu.VMEM_SHARED((NP, HID), _f32),
            pltpu.VMEM((NRELS * NCHUNK, CH), jnp.int32),
            pltpu.VMEM((NRELS * NCHUNK, CH), jnp.int32),
            pltpu.VMEM((CH, HID), _f32),
            pltpu.VMEM((64, HID), _f32),
            pltpu.SemaphoreType.DMA,
        ],
    )


def _message_pass(y0, y1, y2, y3, sidx, didx):
    return _get_sc_msg()(y0, y1, y2, y3, sidx, didx)


def _dot(a, b):
    return jnp.dot(a, b, preferred_element_type=_f32)


def _sigmoid(x):
    return 1.0 / (1.0 + jnp.exp(-x))


def _init_body(xc_ref, tid_ref, temb_ref, WaT_ref, WbT_ref, bin_ref,
               WallT_ref, ball_ref, xi_ref, y0_ref, y1_ref, y2_ref, y3_ref):
    ids = tid_ref[...]
    iot = lax.broadcasted_iota(jnp.int32, (BLK, 128), 1)
    oh = (iot == ids).astype(_f32)
    t = _dot(oh, temb_ref[...])
    xi = _dot(xc_ref[...], WaT_ref[...]) + _dot(t, WbT_ref[...]) + bin_ref[...]
    xi_ref[...] = xi
    yall = _dot(xi, WallT_ref[...]) + ball_ref[...]
    y0_ref[...] = yall[:, 0 * HID:1 * HID]
    y1_ref[...] = yall[:, 1 * HID:2 * HID]
    y2_ref[...] = yall[:, 2 * HID:3 * HID]
    y3_ref[...] = yall[:, 3 * HID:4 * HID]


_rb = pl.BlockSpec((BLK, HID), lambda i: (i, 0))


def _whole(shape):
    return pl.BlockSpec(shape, lambda *a: tuple(0 for _ in shape))


_init_call = pl.pallas_call(
    _init_body,
    grid=(NP // BLK,),
    in_specs=[
        _rb,
        pl.BlockSpec((BLK, 1), lambda i: (i, 0)),
        _whole((128, TYPE_DIM)),
        _whole((W2V, HID)),
        _whole((TYPE_DIM, HID)),
        _whole((1, HID)),
        _whole((HID, NRELS * HID)),
        _whole((1, NRELS * HID)),
    ],
    out_specs=[_rb] * 5,
    out_shape=[jax.ShapeDtypeStruct((NP, HID), _f32)] * 5,
)


def _gru_body(m0_ref, m1_ref, h_ref, WihT_ref, bih_ref, WhhT_ref, bhh_ref,
              WallT_ref, ball_ref, hn_ref, y0_ref, y1_ref, y2_ref, y3_ref):
    m = m0_ref[...] + m1_ref[...]
    h = h_ref[...]
    gi = _dot(m, WihT_ref[...]) + bih_ref[...]
    gh = _dot(h, WhhT_ref[...]) + bhh_ref[...]
    r = _sigmoid(gi[:, :HID] + gh[:, :HID])
    z = _sigmoid(gi[:, HID:2 * HID] + gh[:, HID:2 * HID])
    nc = jnp.tanh(gi[:, 2 * HID:] + r * gh[:, 2 * HID:])
    hn = (1.0 - z) * nc + z * h
    hn_ref[...] = hn
    yall = _dot(hn, WallT_ref[...]) + ball_ref[...]
    y0_ref[...] = yall[:, 0 * HID:1 * HID]
    y1_ref[...] = yall[:, 1 * HID:2 * HID]
    y2_ref[...] = yall[:, 2 * HID:3 * HID]
    y3_ref[...] = yall[:, 3 * HID:4 * HID]


_gru_call = pl.pallas_call(
    _gru_body,
    grid=(NP // BLK,),
    in_specs=[
        _rb, _rb, _rb,
        _whole((HID, 3 * HID)),
        _whole((1, 3 * HID)),
        _whole((HID, 3 * HID)),
        _whole((1, 3 * HID)),
        _whole((HID, NRELS * HID)),
        _whole((1, NRELS * HID)),
    ],
    out_specs=[_rb] * 5,
    out_shape=[jax.ShapeDtypeStruct((NP, HID), _f32)] * 5,
)


_Q = N // 4


def _conv_body(xq_ref, W0T_ref, W1T_ref, W2T_ref, b1_ref, W2cT_ref, b2_ref,
               q_ref):
    X0 = xq_ref[0]
    X1 = xq_ref[1]
    X2 = xq_ref[2]
    X3 = xq_ref[3]
    z256 = jnp.zeros((1, 2 * HID), _f32)
    sdX3 = jnp.concatenate([z256, X3[:-1]], axis=0)
    suX0 = jnp.concatenate([X0[1:], z256], axis=0)
    W0T = W0T_ref[...]
    W1T = W1T_ref[...]
    W2T = W2T_ref[...]
    b1 = b1_ref[...]
    r = jnp.maximum
    Z0 = r(_dot(sdX3, W0T) + _dot(X0, W1T) + _dot(X1, W2T) + b1, 0.0)
    Z1 = r(_dot(X0, W0T) + _dot(X1, W1T) + _dot(X2, W2T) + b1, 0.0)
    Z2 = r(_dot(X1, W0T) + _dot(X2, W1T) + _dot(X3, W2T) + b1, 0.0)
    Z3 = r(_dot(X2, W0T) + _dot(X3, W1T) + _dot(suX0, W2T) + b1, 0.0)
    ninf = jnp.full((1, 64), -jnp.inf, _f32)
    sdZ3 = jnp.concatenate([ninf, Z3[:-1]], axis=0)
    Pe = r(sdZ3, r(Z0, Z1))
    Po = r(Z1, r(Z2, Z3))
    W2cT = W2cT_ref[...]
    b2 = b2_ref[...]
    Ce = r(_dot(Pe, W2cT) + b2, 0.0)
    Co = r(_dot(Po, W2cT) + b2, 0.0)
    q_ref[...] = r(Ce, Co)


_conv_call = pl.pallas_call(
    _conv_body,
    in_specs=[
        _whole((4, _Q, 2 * HID)),
        _whole((2 * HID, 64)),
        _whole((2 * HID, 64)),
        _whole((2 * HID, 64)),
        _whole((1, 64)),
        _whole((64, 64)),
        _whole((1, 64)),
    ],
    out_specs=_whole((_Q, 64)),
    out_shape=jax.ShapeDtypeStruct((_Q, 64), _f32),
)

_FLAT = 64 * _Q
_FCB = _FLAT // 10


def _fc_body(w1_ref, qf_ref, b1_ref, w2_ref, b2_ref, out_ref, acc_ref):
    i = pl.program_id(0)

    @pl.when(i == 0)
    def _():
        acc_ref[...] = jnp.zeros_like(acc_ref)

    acc_ref[...] += _dot(w1_ref[...], qf_ref[...])
    z1 = jnp.maximum(acc_ref[...] + b1_ref[...], 0.0)
    out_ref[...] = _dot(w2_ref[...], z1) + b2_ref[...]


_fc_call = pl.pallas_call(
    _fc_body,
    grid=(_FLAT // _FCB,),
    in_specs=[
        pl.BlockSpec((HID, _FCB), lambda i: (0, i)),
        pl.BlockSpec((_FCB, 1), lambda i: (i, 0)),
        _whole((HID, 1)),
        _whole((2, HID)),
        _whole((2, 1)),
    ],
    out_specs=_whole((2, 1)),
    out_shape=jax.ShapeDtypeStruct((2, 1), _f32),
    scratch_shapes=[pltpu.VMEM((HID, 1), _f32)],
)


def _prep_edges(edge_list):
    srcs, dsts = [], []
    pad = jnp.full((NTILES, EPT_PAD - EPT), TRASH, jnp.int32)
    for e in edge_list:
        e = e.astype(jnp.int32)
        srcs.append(jnp.concatenate([e[0].reshape(NTILES, EPT), pad], axis=1)
                    .reshape(NTILES, NCHUNK, CH))
        dsts.append(jnp.concatenate([e[1].reshape(NTILES, EPT), pad], axis=1)
                    .reshape(NTILES, NCHUNK, CH))
    return jnp.stack(srcs), jnp.stack(dsts)


def kernel(x_code, type_ids, edges_AST, edges_CFG, edges_DFG, edges_NCS,
           type_emb, in_proj_W, in_proj_b, rel_AST_W, rel_AST_b, rel_CFG_W,
           rel_CFG_b, rel_DFG_W, rel_DFG_b, rel_NCS_W, rel_NCS_b, gru_W_ih,
           gru_b_ih, gru_W_hh, gru_b_hh, conv1_W, conv1_b, conv2_W, conv2_b,
           fc1_W, fc1_b, fc2_W, fc2_b):
    xc = jnp.pad(x_code, ((0, NP - N), (0, 0)))
    tid = jnp.pad(type_ids.astype(jnp.int32), (0, NP - N)).reshape(NP, 1)
    temb = jnp.pad(type_emb, ((0, 128 - NUM_TYPES), (0, 0)))
    WaT = in_proj_W[:, :W2V].T
    WbT = in_proj_W[:, W2V:].T
    b_in = in_proj_b.reshape(1, HID)
    WallT = jnp.concatenate(
        [rel_AST_W.T, rel_CFG_W.T, rel_DFG_W.T, rel_NCS_W.T], axis=1)
    b_all = jnp.concatenate(
        [rel_AST_b, rel_CFG_b, rel_DFG_b, rel_NCS_b]).reshape(1, NRELS * HID)
    WihT = gru_W_ih.T
    WhhT = gru_W_hh.T
    bih = gru_b_ih.reshape(1, 3 * HID)
    bhh = gru_b_hh.reshape(1, 3 * HID)
    sidx, didx = _prep_edges([edges_AST, edges_CFG, edges_DFG, edges_NCS])

    xi, y0, y1, y2, y3 = _init_call(xc, tid, temb, WaT, WbT, b_in, WallT, b_all)
    h = xi
    for _ in range(TIME_STEPS):
        m0, m1 = _message_pass(y0, y1, y2, y3, sidx, didx)
        h, y0, y1, y2, y3 = _gru_call(m0, m1, h, WihT, bih, WhhT, bhh,
                                      WallT, b_all)

    X = jnp.concatenate([xi[:N], h[:N]], axis=1)
    Xq = X.reshape(_Q, 4, 2 * HID).transpose(1, 0, 2)
    W0T = conv1_W[:, :, 0].T
    W1T = conv1_W[:, :, 1].T
    W2T = conv1_W[:, :, 2].T
    q = _conv_call(Xq, W0T, W1T, W2T, conv1_b.reshape(1, 64),
                   conv2_W[:, :, 0].T, conv2_b.reshape(1, 64))
    qf = q.T.reshape(_FLAT, 1)
    out = _fc_call(fc1_W, qf, fc1_b.reshape(HID, 1), fc2_W,
                   fc2_b.reshape(2, 1))
    return out.T


# speedup vs baseline: 2.4302x; 1.1070x over previous
import functools

import jax
import jax.numpy as jnp
from jax import lax
from jax.experimental import pallas as pl
from jax.experimental.pallas import tpu as pltpu
from jax.experimental.pallas import tpu_sc as plsc

N = 10000
NP = 10240
E = 80000
W2V = 128
TYPE_DIM = 32
HID = 128
NUM_TYPES = 100
TIME_STEPS = 6
NRELS = 4
BLK = 128

NC, NS = 2, 16
NTILES = NC * NS
EPT = E // NTILES
CH = 128
NCHUNK = (EPT + CH - 1) // CH
EPT_PAD = NCHUNK * CH
TRASH = N
RPT = NP // NS

_f32 = jnp.float32


NBUF = 2


def _sc_body(y0, y1, y2, y3, sidx, didx, m0, m1, acc, sbuf, dbuf,
             gb0, gb1, gs0, gs1, ss0, ss1):
    c = lax.axis_index("c")
    s = lax.axis_index("s")
    g = c * NS + s
    base = s * RPT

    def _zfill(r, _):
        for l in range(HID // 16):
            gb0[r, pl.ds(l * 16, 16)] = jnp.zeros((16,), _f32)
        return 0
    lax.fori_loop(0, 64, _zfill, 0)

    def _zcopy(k, _):
        pltpu.sync_copy(gb0.at[pl.ds(0, 64)], acc.at[pl.ds(base + k * 64, 64)])
        return 0
    lax.fori_loop(0, RPT // 64, _zcopy, 0)

    plsc.subcore_barrier()

    ys = [y0, y1, y2, y3]
    gbufs = [gb0, gb1]
    gsems = [gs0, gs1]
    ssems = [ss0, ss1]
    for r in range(NRELS):
        yr = ys[r]
        pltpu.sync_copy(sidx.at[r, g], sbuf)
        pltpu.sync_copy(didx.at[r, g], dbuf)

        for b in range(NBUF):
            pltpu.async_copy(yr.at[sbuf.at[b]], gbufs[b], gsems[b])

        def _group(jj, _, yr=yr):
            for b in range(NBUF):
                row = jj * NBUF + b
                pltpu.make_async_copy(
                    yr.at[sbuf.at[row]], gbufs[b], gsems[b]).wait()
                pltpu.async_copy(
                    gbufs[b], acc.at[dbuf.at[row]], ssems[b], add=True)

                @pl.when(row + NBUF < NCHUNK)
                def _(b=b, row=row):
                    pltpu.make_async_copy(
                        gbufs[b], acc.at[dbuf.at[row]], ssems[b]).wait()
                    pltpu.async_copy(
                        yr.at[sbuf.at[row + NBUF]], gbufs[b], gsems[b])
            return 0
        lax.fori_loop(0, NCHUNK // NBUF, _group, 0)

        for b in range(NBUF):
            row = NCHUNK - NBUF + b
            pltpu.make_async_copy(
                gbufs[b], acc.at[dbuf.at[row]], ssems[b]).wait()

    plsc.subcore_barrier()

    rows = pl.ds(base, RPT)

    @pl.when(c == 0)
    def _():
        pltpu.sync_copy(acc.at[rows], m0.at[rows])

    @pl.when(c == 1)
    def _():
        pltpu.sync_copy(acc.at[rows], m1.at[rows])


@functools.cache
def _get_sc_msg():
    mesh = plsc.VectorSubcoreMesh(
        core_axis_name="c", subcore_axis_name="s",
        num_cores=NC, num_subcores=NS)
    return pl.kernel(
        _sc_body,
        out_type=[jax.ShapeDtypeStruct((NP, HID), _f32),
                  jax.ShapeDtypeStruct((NP, HID), _f32)],
        mesh=mesh,
        scratch_types=(
            [pltpu.VMEM_SHARED((NP, HID), _f32)]
            + [pltpu.VMEM((NCHUNK, CH), jnp.int32)] * 2
            + [pltpu.VMEM((CH, HID), _f32)] * NBUF
            + [pltpu.SemaphoreType.DMA] * (2 * NBUF)
        ),
    )


def _message_pass(y0, y1, y2, y3, sidx, didx):
    return _get_sc_msg()(y0, y1, y2, y3, sidx, didx)


def _dot(a, b):
    return jnp.dot(a, b, preferred_element_type=_f32)


def _sigmoid(x):
    return 1.0 / (1.0 + jnp.exp(-x))


def _init_body(xc_ref, tid_ref, temb_ref, WaT_ref, WbT_ref, bin_ref,
               WallT_ref, ball_ref, xi_ref, y0_ref, y1_ref, y2_ref, y3_ref):
    ids = tid_ref[...]
    iot = lax.broadcasted_iota(jnp.int32, (BLK, 128), 1)
    oh = (iot == ids).astype(_f32)
    t = _dot(oh, temb_ref[...])
    xi = _dot(xc_ref[...], WaT_ref[...]) + _dot(t, WbT_ref[...]) + bin_ref[...]
    xi_ref[...] = xi
    yall = _dot(xi, WallT_ref[...]) + ball_ref[...]
    y0_ref[...] = yall[:, 0 * HID:1 * HID]
    y1_ref[...] = yall[:, 1 * HID:2 * HID]
    y2_ref[...] = yall[:, 2 * HID:3 * HID]
    y3_ref[...] = yall[:, 3 * HID:4 * HID]


_rb = pl.BlockSpec((BLK, HID), lambda i: (i, 0))


def _whole(shape):
    return pl.BlockSpec(shape, lambda *a: tuple(0 for _ in shape))


_init_call = pl.pallas_call(
    _init_body,
    grid=(NP // BLK,),
    in_specs=[
        _rb,
        pl.BlockSpec((BLK, 1), lambda i: (i, 0)),
        _whole((128, TYPE_DIM)),
        _whole((W2V, HID)),
        _whole((TYPE_DIM, HID)),
        _whole((1, HID)),
        _whole((HID, NRELS * HID)),
        _whole((1, NRELS * HID)),
    ],
    out_specs=[_rb] * 5,
    out_shape=[jax.ShapeDtypeStruct((NP, HID), _f32)] * 5,
)


def _gru_body(m0_ref, m1_ref, h_ref, WihT_ref, bih_ref, WhhT_ref, bhh_ref,
              WallT_ref, ball_ref, hn_ref, y0_ref, y1_ref, y2_ref, y3_ref):
    m = m0_ref[...] + m1_ref[...]
    h = h_ref[...]
    gi = _dot(m, WihT_ref[...]) + bih_ref[...]
    gh = _dot(h, WhhT_ref[...]) + bhh_ref[...]
    r = _sigmoid(gi[:, :HID] + gh[:, :HID])
    z = _sigmoid(gi[:, HID:2 * HID] + gh[:, HID:2 * HID])
    nc = jnp.tanh(gi[:, 2 * HID:] + r * gh[:, 2 * HID:])
    hn = (1.0 - z) * nc + z * h
    hn_ref[...] = hn
    yall = _dot(hn, WallT_ref[...]) + ball_ref[...]
    y0_ref[...] = yall[:, 0 * HID:1 * HID]
    y1_ref[...] = yall[:, 1 * HID:2 * HID]
    y2_ref[...] = yall[:, 2 * HID:3 * HID]
    y3_ref[...] = yall[:, 3 * HID:4 * HID]


_gru_call = pl.pallas_call(
    _gru_body,
    grid=(NP // BLK,),
    in_specs=[
        _rb, _rb, _rb,
        _whole((HID, 3 * HID)),
        _whole((1, 3 * HID)),
        _whole((HID, 3 * HID)),
        _whole((1, 3 * HID)),
        _whole((HID, NRELS * HID)),
        _whole((1, NRELS * HID)),
    ],
    out_specs=[_rb] * 5,
    out_shape=[jax.ShapeDtypeStruct((NP, HID), _f32)] * 5,
)


_Q = N // 4


def _conv_body(xq_ref, W0T_ref, W1T_ref, W2T_ref, b1_ref, W2cT_ref, b2_ref,
               q_ref):
    X0 = xq_ref[0]
    X1 = xq_ref[1]
    X2 = xq_ref[2]
    X3 = xq_ref[3]
    z256 = jnp.zeros((1, 2 * HID), _f32)
    sdX3 = jnp.concatenate([z256, X3[:-1]], axis=0)
    suX0 = jnp.concatenate([X0[1:], z256], axis=0)
    W0T = W0T_ref[...]
    W1T = W1T_ref[...]
    W2T = W2T_ref[...]
    b1 = b1_ref[...]
    r = jnp.maximum
    Z0 = r(_dot(sdX3, W0T) + _dot(X0, W1T) + _dot(X1, W2T) + b1, 0.0)
    Z1 = r(_dot(X0, W0T) + _dot(X1, W1T) + _dot(X2, W2T) + b1, 0.0)
    Z2 = r(_dot(X1, W0T) + _dot(X2, W1T) + _dot(X3, W2T) + b1, 0.0)
    Z3 = r(_dot(X2, W0T) + _dot(X3, W1T) + _dot(suX0, W2T) + b1, 0.0)
    ninf = jnp.full((1, 64), -jnp.inf, _f32)
    sdZ3 = jnp.concatenate([ninf, Z3[:-1]], axis=0)
    Pe = r(sdZ3, r(Z0, Z1))
    Po = r(Z1, r(Z2, Z3))
    W2cT = W2cT_ref[...]
    b2 = b2_ref[...]
    Ce = r(_dot(Pe, W2cT) + b2, 0.0)
    Co = r(_dot(Po, W2cT) + b2, 0.0)
    q_ref[...] = r(Ce, Co)


_conv_call = pl.pallas_call(
    _conv_body,
    in_specs=[
        _whole((4, _Q, 2 * HID)),
        _whole((2 * HID, 64)),
        _whole((2 * HID, 64)),
        _whole((2 * HID, 64)),
        _whole((1, 64)),
        _whole((64, 64)),
        _whole((1, 64)),
    ],
    out_specs=_whole((_Q, 64)),
    out_shape=jax.ShapeDtypeStruct((_Q, 64), _f32),
)

_FLAT = 64 * _Q
_FCB = _FLAT // 10


def _fc_body(w1_ref, qf_ref, b1_ref, w2_ref, b2_ref, out_ref, acc_ref):
    i = pl.program_id(0)

    @pl.when(i == 0)
    def _():
        acc_ref[...] = jnp.zeros_like(acc_ref)

    acc_ref[...] += _dot(w1_ref[...], qf_ref[...])
    z1 = jnp.maximum(acc_ref[...] + b1_ref[...], 0.0)
    out_ref[...] = _dot(w2_ref[...], z1) + b2_ref[...]


_fc_call = pl.pallas_call(
    _fc_body,
    grid=(_FLAT // _FCB,),
    in_specs=[
        pl.BlockSpec((HID, _FCB), lambda i: (0, i)),
        pl.BlockSpec((_FCB, 1), lambda i: (i, 0)),
        _whole((HID, 1)),
        _whole((2, HID)),
        _whole((2, 1)),
    ],
    out_specs=_whole((2, 1)),
    out_shape=jax.ShapeDtypeStruct((2, 1), _f32),
    scratch_shapes=[pltpu.VMEM((HID, 1), _f32)],
)


def _prep_edges(edge_list):
    srcs, dsts = [], []
    pad = jnp.full((NTILES, EPT_PAD - EPT), TRASH, jnp.int32)
    for e in edge_list:
        e = e.astype(jnp.int32)
        srcs.append(jnp.concatenate([e[0].reshape(NTILES, EPT), pad], axis=1)
                    .reshape(NTILES, NCHUNK, CH))
        dsts.append(jnp.concatenate([e[1].reshape(NTILES, EPT), pad], axis=1)
                    .reshape(NTILES, NCHUNK, CH))
    return jnp.stack(srcs), jnp.stack(dsts)


def kernel(x_code, type_ids, edges_AST, edges_CFG, edges_DFG, edges_NCS,
           type_emb, in_proj_W, in_proj_b, rel_AST_W, rel_AST_b, rel_CFG_W,
           rel_CFG_b, rel_DFG_W, rel_DFG_b, rel_NCS_W, rel_NCS_b, gru_W_ih,
           gru_b_ih, gru_W_hh, gru_b_hh, conv1_W, conv1_b, conv2_W, conv2_b,
           fc1_W, fc1_b, fc2_W, fc2_b):
    xc = jnp.pad(x_code, ((0, NP - N), (0, 0)))
    tid = jnp.pad(type_ids.astype(jnp.int32), (0, NP - N)).reshape(NP, 1)
    temb = jnp.pad(type_emb, ((0, 128 - NUM_TYPES), (0, 0)))
    WaT = in_proj_W[:, :W2V].T
    WbT = in_proj_W[:, W2V:].T
    b_in = in_proj_b.reshape(1, HID)
    WallT = jnp.concatenate(
        [rel_AST_W.T, rel_CFG_W.T, rel_DFG_W.T, rel_NCS_W.T], axis=1)
    b_all = jnp.concatenate(
        [rel_AST_b, rel_CFG_b, rel_DFG_b, rel_NCS_b]).reshape(1, NRELS * HID)
    WihT = gru_W_ih.T
    WhhT = gru_W_hh.T
    bih = gru_b_ih.reshape(1, 3 * HID)
    bhh = gru_b_hh.reshape(1, 3 * HID)
    sidx, didx = _prep_edges([edges_AST, edges_CFG, edges_DFG, edges_NCS])

    xi, y0, y1, y2, y3 = _init_call(xc, tid, temb, WaT, WbT, b_in, WallT, b_all)
    h = xi
    for _ in range(TIME_STEPS):
        m0, m1 = _message_pass(y0, y1, y2, y3, sidx, didx)
        h, y0, y1, y2, y3 = _gru_call(m0, m1, h, WihT, bih, WhhT, bhh,
                                      WallT, b_all)

    X = jnp.concatenate([xi[:N], h[:N]], axis=1)
    Xq = X.reshape(_Q, 4, 2 * HID).transpose(1, 0, 2)
    W0T = conv1_W[:, :, 0].T
    W1T = conv1_W[:, :, 1].T
    W2T = conv1_W[:, :, 2].T
    q = _conv_call(Xq, W0T, W1T, W2T, conv1_b.reshape(1, 64),
                   conv2_W[:, :, 0].T, conv2_b.reshape(1, 64))
    qf = q.T.reshape(_FLAT, 1)
    out = _fc_call(fc1_W, qf, fc1_b.reshape(HID, 1), fc2_W,
                   fc2_b.reshape(2, 1))
    return out.T


# speedup vs baseline: 2.4423x; 1.0050x over previous
import functools

import jax
import jax.numpy as jnp
from jax import lax
from jax.experimental import pallas as pl
from jax.experimental.pallas import tpu as pltpu
from jax.experimental.pallas import tpu_sc as plsc

N = 10000
NP = 10240
E = 80000
W2V = 128
TYPE_DIM = 32
HID = 128
NUM_TYPES = 100
TIME_STEPS = 6
NRELS = 4
BLK = 128

NC, NS = 2, 16
NTILES = NC * NS
EPT = E // NTILES
CH = 64
NCHUNK = (EPT + CH - 1) // CH
EPT_PAD = NCHUNK * CH
TRASH = N
RPT = NP // NS

_f32 = jnp.float32


NBUF = 4


def _sc_body(y0, y1, y2, y3, sidx, didx, m0, m1, acc, sbuf, dbuf,
             gb0, gb1, gb2, gb3, gs0, gs1, gs2, gs3, ss0, ss1, ss2, ss3):
    c = lax.axis_index("c")
    s = lax.axis_index("s")
    g = c * NS + s
    base = s * RPT

    def _zfill(r, _):
        for l in range(HID // 16):
            gb0[r, pl.ds(l * 16, 16)] = jnp.zeros((16,), _f32)
        return 0
    lax.fori_loop(0, CH, _zfill, 0)

    def _zcopy(k, _):
        pltpu.sync_copy(gb0.at[pl.ds(0, CH)], acc.at[pl.ds(base + k * CH, CH)])
        return 0
    lax.fori_loop(0, RPT // CH, _zcopy, 0)

    plsc.subcore_barrier()

    ys = [y0, y1, y2, y3]
    gbufs = [gb0, gb1, gb2, gb3]
    gsems = [gs0, gs1, gs2, gs3]
    ssems = [ss0, ss1, ss2, ss3]
    for r in range(NRELS):
        yr = ys[r]
        pltpu.sync_copy(sidx.at[r, g], sbuf)
        pltpu.sync_copy(didx.at[r, g], dbuf)

        for b in range(NBUF):
            pltpu.async_copy(yr.at[sbuf.at[b]], gbufs[b], gsems[b])

        def _group(jj, _, yr=yr):
            for b in range(NBUF):
                row = jj * NBUF + b
                pltpu.make_async_copy(
                    yr.at[sbuf.at[row]], gbufs[b], gsems[b]).wait()
                pltpu.async_copy(
                    gbufs[b], acc.at[dbuf.at[row]], ssems[b], add=True)

                @pl.when(row + NBUF < NCHUNK)
                def _(b=b, row=row):
                    pltpu.make_async_copy(
                        gbufs[b], acc.at[dbuf.at[row]], ssems[b]).wait()
                    pltpu.async_copy(
                        yr.at[sbuf.at[row + NBUF]], gbufs[b], gsems[b])
            return 0
        lax.fori_loop(0, NCHUNK // NBUF, _group, 0)

        for b in range(NBUF):
            row = NCHUNK - NBUF + b
            pltpu.make_async_copy(
                gbufs[b], acc.at[dbuf.at[row]], ssems[b]).wait()

    plsc.subcore_barrier()

    rows = pl.ds(base, RPT)

    @pl.when(c == 0)
    def _():
        pltpu.sync_copy(acc.at[rows], m0.at[rows])

    @pl.when(c == 1)
    def _():
        pltpu.sync_copy(acc.at[rows], m1.at[rows])


@functools.cache
def _get_sc_msg():
    mesh = plsc.VectorSubcoreMesh(
        core_axis_name="c", subcore_axis_name="s",
        num_cores=NC, num_subcores=NS)
    return pl.kernel(
        _sc_body,
        out_type=[jax.ShapeDtypeStruct((NP, HID), _f32),
                  jax.ShapeDtypeStruct((NP, HID), _f32)],
        mesh=mesh,
        scratch_types=(
            [pltpu.VMEM_SHARED((NP, HID), _f32)]
            + [pltpu.VMEM((NCHUNK, CH), jnp.int32)] * 2
            + [pltpu.VMEM((CH, HID), _f32)] * NBUF
            + [pltpu.SemaphoreType.DMA] * (2 * NBUF)
        ),
    )


def _message_pass(y0, y1, y2, y3, sidx, didx):
    return _get_sc_msg()(y0, y1, y2, y3, sidx, didx)


def _dot(a, b):
    return jnp.dot(a, b, preferred_element_type=_f32)


def _sigmoid(x):
    return 1.0 / (1.0 + jnp.exp(-x))


def _init_body(xc_ref, tid_ref, temb_ref, WaT_ref, WbT_ref, bin_ref,
               WallT_ref, ball_ref, xi_ref, y0_ref, y1_ref, y2_ref, y3_ref):
    ids = tid_ref[...]
    iot = lax.broadcasted_iota(jnp.int32, (BLK, 128), 1)
    oh = (iot == ids).astype(_f32)
    t = _dot(oh, temb_ref[...])
    xi = _dot(xc_ref[...], WaT_ref[...]) + _dot(t, WbT_ref[...]) + bin_ref[...]
    xi_ref[...] = xi
    yall = _dot(xi, WallT_ref[...]) + ball_ref[...]
    y0_ref[...] = yall[:, 0 * HID:1 * HID]
    y1_ref[...] = yall[:, 1 * HID:2 * HID]
    y2_ref[...] = yall[:, 2 * HID:3 * HID]
    y3_ref[...] = yall[:, 3 * HID:4 * HID]


_rb = pl.BlockSpec((BLK, HID), lambda i: (i, 0))


def _whole(shape):
    return pl.BlockSpec(shape, lambda *a: tuple(0 for _ in shape))


_init_call = pl.pallas_call(
    _init_body,
    grid=(NP // BLK,),
    in_specs=[
        _rb,
        pl.BlockSpec((BLK, 1), lambda i: (i, 0)),
        _whole((128, TYPE_DIM)),
        _whole((W2V, HID)),
        _whole((TYPE_DIM, HID)),
        _whole((1, HID)),
        _whole((HID, NRELS * HID)),
        _whole((1, NRELS * HID)),
    ],
    out_specs=[_rb] * 5,
    out_shape=[jax.ShapeDtypeStruct((NP, HID), _f32)] * 5,
)


def _gru_body(m0_ref, m1_ref, h_ref, WihT_ref, bih_ref, WhhT_ref, bhh_ref,
              WallT_ref, ball_ref, hn_ref, y0_ref, y1_ref, y2_ref, y3_ref):
    m = m0_ref[...] + m1_ref[...]
    h = h_ref[...]
    gi = _dot(m, WihT_ref[...]) + bih_ref[...]
    gh = _dot(h, WhhT_ref[...]) + bhh_ref[...]
    r = _sigmoid(gi[:, :HID] + gh[:, :HID])
    z = _sigmoid(gi[:, HID:2 * HID] + gh[:, HID:2 * HID])
    nc = jnp.tanh(gi[:, 2 * HID:] + r * gh[:, 2 * HID:])
    hn = (1.0 - z) * nc + z * h
    hn_ref[...] = hn
    yall = _dot(hn, WallT_ref[...]) + ball_ref[...]
    y0_ref[...] = yall[:, 0 * HID:1 * HID]
    y1_ref[...] = yall[:, 1 * HID:2 * HID]
    y2_ref[...] = yall[:, 2 * HID:3 * HID]
    y3_ref[...] = yall[:, 3 * HID:4 * HID]


_gru_call = pl.pallas_call(
    _gru_body,
    grid=(NP // BLK,),
    in_specs=[
        _rb, _rb, _rb,
        _whole((HID, 3 * HID)),
        _whole((1, 3 * HID)),
        _whole((HID, 3 * HID)),
        _whole((1, 3 * HID)),
        _whole((HID, NRELS * HID)),
        _whole((1, NRELS * HID)),
    ],
    out_specs=[_rb] * 5,
    out_shape=[jax.ShapeDtypeStruct((NP, HID), _f32)] * 5,
)


_Q = N // 4


def _conv_body(xq_ref, W0T_ref, W1T_ref, W2T_ref, b1_ref, W2cT_ref, b2_ref,
               q_ref):
    X0 = xq_ref[0]
    X1 = xq_ref[1]
    X2 = xq_ref[2]
    X3 = xq_ref[3]
    z256 = jnp.zeros((1, 2 * HID), _f32)
    sdX3 = jnp.concatenate([z256, X3[:-1]], axis=0)
    suX0 = jnp.concatenate([X0[1:], z256], axis=0)
    W0T = W0T_ref[...]
    W1T = W1T_ref[...]
    W2T = W2T_ref[...]
    b1 = b1_ref[...]
    r = jnp.maximum
    Z0 = r(_dot(sdX3, W0T) + _dot(X0, W1T) + _dot(X1, W2T) + b1, 0.0)
    Z1 = r(_dot(X0, W0T) + _dot(X1, W1T) + _dot(X2, W2T) + b1, 0.0)
    Z2 = r(_dot(X1, W0T) + _dot(X2, W1T) + _dot(X3, W2T) + b1, 0.0)
    Z3 = r(_dot(X2, W0T) + _dot(X3, W1T) + _dot(suX0, W2T) + b1, 0.0)
    ninf = jnp.full((1, 64), -jnp.inf, _f32)
    sdZ3 = jnp.concatenate([ninf, Z3[:-1]], axis=0)
    Pe = r(sdZ3, r(Z0, Z1))
    Po = r(Z1, r(Z2, Z3))
    W2cT = W2cT_ref[...]
    b2 = b2_ref[...]
    Ce = r(_dot(Pe, W2cT) + b2, 0.0)
    Co = r(_dot(Po, W2cT) + b2, 0.0)
    q_ref[...] = r(Ce, Co)


_conv_call = pl.pallas_call(
    _conv_body,
    in_specs=[
        _whole((4, _Q, 2 * HID)),
        _whole((2 * HID, 64)),
        _whole((2 * HID, 64)),
        _whole((2 * HID, 64)),
        _whole((1, 64)),
        _whole((64, 64)),
        _whole((1, 64)),
    ],
    out_specs=_whole((_Q, 64)),
    out_shape=jax.ShapeDtypeStruct((_Q, 64), _f32),
)

_FLAT = 64 * _Q
_FCB = _FLAT // 10


def _fc_body(w1_ref, qf_ref, b1_ref, w2_ref, b2_ref, out_ref, acc_ref):
    i = pl.program_id(0)

    @pl.when(i == 0)
    def _():
        acc_ref[...] = jnp.zeros_like(acc_ref)

    acc_ref[...] += _dot(w1_ref[...], qf_ref[...])
    z1 = jnp.maximum(acc_ref[...] + b1_ref[...], 0.0)
    out_ref[...] = _dot(w2_ref[...], z1) + b2_ref[...]


_fc_call = pl.pallas_call(
    _fc_body,
    grid=(_FLAT // _FCB,),
    in_specs=[
        pl.BlockSpec((HID, _FCB), lambda i: (0, i)),
        pl.BlockSpec((_FCB, 1), lambda i: (i, 0)),
        _whole((HID, 1)),
        _whole((2, HID)),
        _whole((2, 1)),
    ],
    out_specs=_whole((2, 1)),
    out_shape=jax.ShapeDtypeStruct((2, 1), _f32),
    scratch_shapes=[pltpu.VMEM((HID, 1), _f32)],
)


def _prep_edges(edge_list):
    srcs, dsts = [], []
    pad = jnp.full((NTILES, EPT_PAD - EPT), TRASH, jnp.int32)
    for e in edge_list:
        e = e.astype(jnp.int32)
        srcs.append(jnp.concatenate([e[0].reshape(NTILES, EPT), pad], axis=1)
                    .reshape(NTILES, NCHUNK, CH))
        dsts.append(jnp.concatenate([e[1].reshape(NTILES, EPT), pad], axis=1)
                    .reshape(NTILES, NCHUNK, CH))
    return jnp.stack(srcs), jnp.stack(dsts)


def kernel(x_code, type_ids, edges_AST, edges_CFG, edges_DFG, edges_NCS,
           type_emb, in_proj_W, in_proj_b, rel_AST_W, rel_AST_b, rel_CFG_W,
           rel_CFG_b, rel_DFG_W, rel_DFG_b, rel_NCS_W, rel_NCS_b, gru_W_ih,
           gru_b_ih, gru_W_hh, gru_b_hh, conv1_W, conv1_b, conv2_W, conv2_b,
           fc1_W, fc1_b, fc2_W, fc2_b):
    xc = jnp.pad(x_code, ((0, NP - N), (0, 0)))
    tid = jnp.pad(type_ids.astype(jnp.int32), (0, NP - N)).reshape(NP, 1)
    temb = jnp.pad(type_emb, ((0, 128 - NUM_TYPES), (0, 0)))
    WaT = in_proj_W[:, :W2V].T
    WbT = in_proj_W[:, W2V:].T
    b_in = in_proj_b.reshape(1, HID)
    WallT = jnp.concatenate(
        [rel_AST_W.T, rel_CFG_W.T, rel_DFG_W.T, rel_NCS_W.T], axis=1)
    b_all = jnp.concatenate(
        [rel_AST_b, rel_CFG_b, rel_DFG_b, rel_NCS_b]).reshape(1, NRELS * HID)
    WihT = gru_W_ih.T
    WhhT = gru_W_hh.T
    bih = gru_b_ih.reshape(1, 3 * HID)
    bhh = gru_b_hh.reshape(1, 3 * HID)
    sidx, didx = _prep_edges([edges_AST, edges_CFG, edges_DFG, edges_NCS])

    xi, y0, y1, y2, y3 = _init_call(xc, tid, temb, WaT, WbT, b_in, WallT, b_all)
    h = xi
    for _ in range(TIME_STEPS):
        m0, m1 = _message_pass(y0, y1, y2, y3, sidx, didx)
        h, y0, y1, y2, y3 = _gru_call(m0, m1, h, WihT, bih, WhhT, bhh,
                                      WallT, b_all)

    X = jnp.concatenate([xi[:N], h[:N]], axis=1)
    Xq = X.reshape(_Q, 4, 2 * HID).transpose(1, 0, 2)
    W0T = conv1_W[:, :, 0].T
    W1T = conv1_W[:, :, 1].T
    W2T = conv1_W[:, :, 2].T
    q = _conv_call(Xq, W0T, W1T, W2T, conv1_b.reshape(1, 64),
                   conv2_W[:, :, 0].T, conv2_b.reshape(1, 64))
    qf = q.T.reshape(_FLAT, 1)
    out = _fc_call(fc1_W, qf, fc1_b.reshape(HID, 1), fc2_W,
                   fc2_b.reshape(2, 1))
    return out.T
